# Initial kernel scaffold; baseline (speedup 1.0000x reference)
import jax
import jax.numpy as jnp
from jax.experimental import pallas as pl


def kernel(x, pe):
    raise NotImplementedError("write your pallas kernel here")


# speedup vs baseline: 5.0231x; 5.0231x over previous
import jax
import jax.numpy as jnp
from jax.experimental import pallas as pl

_BS = 512


def _body(pe_ref, out_ref):
    out_ref[...] = jnp.broadcast_to(pe_ref[...][None], out_ref.shape)


def kernel(x, pe):
    B, S = x.shape
    _, E = pe.shape
    return pl.pallas_call(
        _body,
        grid=(S // _BS,),
        in_specs=[pl.BlockSpec((_BS, E), lambda i: (i, 0))],
        out_specs=pl.BlockSpec((B, _BS, E), lambda i: (0, i, 0)),
        out_shape=jax.ShapeDtypeStruct((B, S, E), pe.dtype),
    )(pe)
